# baseline (speedup 1.0000x reference)
import functools

import jax
import jax.numpy as jnp
from jax import lax
from jax.experimental import pallas as pl
from jax.experimental.pallas import tpu as pltpu
from jax.experimental.pallas import tpu_sc as plsc

EMBED_DIM = 32
BATCH = 16384
LANES = 16
PACK = 128 // EMBED_DIM

NUM_CORES = 2
NUM_SUBCORES = 16
NUM_WORKERS = NUM_CORES * NUM_SUBCORES
B_PER_W = BATCH // NUM_WORKERS
CHUNK = 128
HALF = B_PER_W // 2
N_CHUNK = HALF // CHUNK


def _dot_body(uids_hbm, iids_hbm, utab_hbm, itab_hbm, out_hbm,
              uid_v, iid_v, ulid_v, ilid_v, ulines, ilines, out_v, sem):
    wid = lax.axis_index("s") * NUM_CORES + lax.axis_index("c")
    base = wid * B_PER_W
    idx_row = wid * (B_PER_W // CHUNK)

    pltpu.sync_copy(uids_hbm.at[pl.ds(idx_row, B_PER_W // CHUNK)], uid_v)
    pltpu.sync_copy(iids_hbm.at[pl.ds(idx_row, B_PER_W // CHUNK)], iid_v)

    for j in range(B_PER_W // CHUNK):
        for k in range(CHUNK // LANES):
            s = pl.ds(k * LANES, LANES)
            ulid_v[j, s] = jax.lax.shift_right_logical(uid_v[j, s], 2)
            ilid_v[j, s] = jax.lax.shift_right_logical(iid_v[j, s], 2)

    lane = lax.iota(jnp.int32, LANES)

    for h in range(2):
        copies = []
        for j in range(N_CHUNK):
            jj = h * N_CHUNK + j
            copies.append(pltpu.async_copy(
                utab_hbm.at[ulid_v.at[jj]],
                ulines.at[pl.ds(j * CHUNK, CHUNK)], sem))
            copies.append(pltpu.async_copy(
                itab_hbm.at[ilid_v.at[jj]],
                ilines.at[pl.ds(j * CHUNK, CHUNK)], sem))
        for c in copies:
            c.wait()

        def body(g, _):
            b0 = g * LANES
            rows = b0 + lane
            jj = h * N_CHUNK + b0 // CHUNK
            s = pl.ds(b0 % CHUNK, LANES)
            uoff = (uid_v[jj, s] & (PACK - 1)) * EMBED_DIM
            ioff = (iid_v[jj, s] & (PACK - 1)) * EMBED_DIM
            acc = jnp.zeros((LANES,), jnp.float32)
            for d in range(EMBED_DIM):
                uc = plsc.load_gather(ulines, [rows, uoff + d])
                vc = plsc.load_gather(ilines, [rows, ioff + d])
                acc = acc + uc * vc
            out_v[pl.ds(h * HALF + b0, LANES)] = acc
            return 0

        lax.fori_loop(0, HALF // LANES, body, 0)

    pltpu.sync_copy(out_v, out_hbm.at[pl.ds(base, B_PER_W)])


_sc_call = functools.partial(
    pl.kernel,
    mesh=plsc.VectorSubcoreMesh(core_axis_name="c", subcore_axis_name="s"),
    out_type=jax.ShapeDtypeStruct((BATCH,), jnp.float32),
    compiler_params=pltpu.CompilerParams(needs_layout_passes=False),
    scratch_types=[
        pltpu.VMEM((B_PER_W // CHUNK, CHUNK), jnp.int32),
        pltpu.VMEM((B_PER_W // CHUNK, CHUNK), jnp.int32),
        pltpu.VMEM((B_PER_W // CHUNK, CHUNK), jnp.int32),
        pltpu.VMEM((B_PER_W // CHUNK, CHUNK), jnp.int32),
        pltpu.VMEM((HALF, CHUNK), jnp.float32),
        pltpu.VMEM((HALF, CHUNK), jnp.float32),
        pltpu.VMEM((B_PER_W,), jnp.float32),
        pltpu.SemaphoreType.DMA,
    ],
)(_dot_body)


@jax.jit
def kernel(user_ids, item_ids, user_table, item_table):
    uids = jnp.asarray(user_ids, jnp.int32).reshape(BATCH // CHUNK, CHUNK)
    iids = jnp.asarray(item_ids, jnp.int32).reshape(BATCH // CHUNK, CHUNK)
    utab = user_table.reshape(-1, CHUNK)
    itab = item_table.reshape(-1, CHUNK)
    return _sc_call(uids, iids, utab, itab)
